# baseline (speedup 1.0000x reference)
import functools
import jax
import jax.numpy as jnp
from jax import lax
from jax.experimental import pallas as pl
from jax.experimental.pallas import tpu as pltpu
from jax.experimental.pallas import tpu_sc as plsc

B = 16384
D = 64
NC = 2
NS = 16
NW = NC * NS
EPW = B // NW
SUB = 8
NSUB = EPW // SUB
LANES = 16
NJC = D // LANES


def _rsqrt_newton(x):
    xi = lax.bitcast_convert_type(x, jnp.int32)
    yi = jnp.full((LANES,), 0x5F3759DF, jnp.int32) - lax.shift_right_arithmetic(
        xi, jnp.full((LANES,), 1, jnp.int32))
    y = lax.bitcast_convert_type(yi, jnp.float32)
    xh = x * jnp.float32(0.5)
    for _ in range(3):
        y = y * (jnp.float32(1.5) - xh * y * y)
    return y


def _make_sc_kernel():
    mesh = plsc.VectorSubcoreMesh(core_axis_name="c", subcore_axis_name="s")

    @functools.partial(
        pl.kernel,
        out_type=jax.ShapeDtypeStruct((NW, LANES), jnp.float32),
        mesh=mesh,
        compiler_params=pltpu.CompilerParams(use_tc_tiling_on_sc=False),
        scratch_types=[
            pltpu.VMEM((EPW,), jnp.int32),
            pltpu.VMEM((EPW,), jnp.int32),
            pltpu.VMEM((EPW,), jnp.int32),
            pltpu.VMEM((EPW,), jnp.int32),
            pltpu.VMEM((EPW,), jnp.int32),
            pltpu.VMEM((EPW,), jnp.int32),
            pltpu.VMEM((EPW,), jnp.int32),
            pltpu.VMEM((SUB, D), jnp.float32),
            pltpu.VMEM((SUB, D), jnp.float32),
            pltpu.VMEM((SUB, D), jnp.float32),
            pltpu.VMEM((SUB, D), jnp.float32),
            pltpu.VMEM((SUB, D), jnp.float32),
            pltpu.VMEM((SUB, D), jnp.float32),
            pltpu.VMEM((SUB, D * D), jnp.float32),
            pltpu.VMEM((LANES,), jnp.float32),
        ],
    )
    def k(pos_h, pos_t, pos_r, pos_type_r, neg_h, neg_t, neg_r,
          ent, rel, mat, out_hbm,
          iph, ipt, inh, int_, ipr, inr, ity,
          rph, rpt, rnh, rnt, rpr, rnr, mbuf, outv):
        wid = lax.axis_index("s") * NC + lax.axis_index("c")
        base = wid * EPW

        pltpu.sync_copy(pos_h.at[pl.ds(base, EPW)], iph)
        pltpu.sync_copy(pos_t.at[pl.ds(base, EPW)], ipt)
        pltpu.sync_copy(neg_h.at[pl.ds(base, EPW)], inh)
        pltpu.sync_copy(neg_t.at[pl.ds(base, EPW)], int_)
        pltpu.sync_copy(pos_r.at[pl.ds(base, EPW)], ipr)
        pltpu.sync_copy(neg_r.at[pl.ds(base, EPW)], inr)
        pltpu.sync_copy(pos_type_r.at[pl.ds(base, EPW)], ity)

        lane = lax.iota(jnp.int32, LANES)
        perms = [lane ^ k for k in (8, 4, 2, 1)]

        def allsum(v):
            for p in perms:
                v = v + v.at[p].get(mode="promise_in_bounds",
                                    unique_indices=True)
            return v

        def norm_scale(chunks):
            ss = chunks[0] * chunks[0]
            for c in chunks[1:]:
                ss = ss + c * c
            tot = allsum(ss)
            return _rsqrt_newton(jnp.maximum(tot, jnp.float32(1e-12)))

        def element(e):
            def body(ci, accs):
                hch = rph[e, pl.ds(ci * LANES, LANES)]
                tch = rpt[e, pl.ds(ci * LANES, LANES)]
                nhch = rnh[e, pl.ds(ci * LANES, LANES)]
                ntch = rnt[e, pl.ds(ci * LANES, LANES)]
                a0, a1, a2, a3 = accs
                a0, a1, a2, a3 = list(a0), list(a1), list(a2), list(a3)
                for li in range(LANES):
                    sh = hch[li]
                    st = tch[li]
                    snh = nhch[li]
                    snt = ntch[li]
                    moff = ci * (LANES * D) + li * D
                    for jc in range(NJC):
                        m = mbuf[e, pl.ds(moff + jc * LANES, LANES)]
                        a0[jc] = a0[jc] + sh * m
                        a1[jc] = a1[jc] + st * m
                        a2[jc] = a2[jc] + snh * m
                        a3[jc] = a3[jc] + snt * m
                return (tuple(a0), tuple(a1), tuple(a2), tuple(a3))

            zero = jnp.zeros((LANES,), jnp.float32)
            init = tuple(tuple(zero for _ in range(NJC)) for _ in range(4))
            h_, t_, nh_, nt_ = lax.fori_loop(0, D // LANES, body, init)

            pr = tuple(rpr[e, pl.ds(jc * LANES, LANES)] for jc in range(NJC))
            nr = tuple(rnr[e, pl.ds(jc * LANES, LANES)] for jc in range(NJC))

            a_h = norm_scale(h_)
            a_t = norm_scale(t_)
            a_nh = norm_scale(nh_)
            a_nt = norm_scale(nt_)
            a_pr = norm_scale(pr)
            a_nr = norm_scale(nr)

            ps = jnp.zeros((LANES,), jnp.float32)
            ns = jnp.zeros((LANES,), jnp.float32)
            for jc in range(NJC):
                ps = ps + jnp.abs(h_[jc] * a_h + pr[jc] * a_pr - t_[jc] * a_t)
                ns = ns + jnp.abs(nh_[jc] * a_nh + nr[jc] * a_nr - nt_[jc] * a_nt)
            diff = allsum(ps - ns)
            return jnp.maximum(diff + jnp.float32(1.0), jnp.float32(0.0))

        def chunk_body(c, loss_acc):
            off = pl.multiple_of(c * SUB, SUB)
            pltpu.sync_copy(ent.at[iph.at[pl.ds(off, SUB)]], rph)
            pltpu.sync_copy(ent.at[ipt.at[pl.ds(off, SUB)]], rpt)
            pltpu.sync_copy(ent.at[inh.at[pl.ds(off, SUB)]], rnh)
            pltpu.sync_copy(ent.at[int_.at[pl.ds(off, SUB)]], rnt)
            pltpu.sync_copy(rel.at[ipr.at[pl.ds(off, SUB)]], rpr)
            pltpu.sync_copy(rel.at[inr.at[pl.ds(off, SUB)]], rnr)
            pltpu.sync_copy(mat.at[ity.at[pl.ds(off, SUB)]], mbuf)
            return lax.fori_loop(
                0, SUB, lambda e, acc: acc + element(e), loss_acc)

        loss = lax.fori_loop(0, NSUB, chunk_body,
                             jnp.zeros((LANES,), jnp.float32))

        outv[...] = loss
        pltpu.sync_copy(outv, out_hbm.at[wid])

    return k


_sc_kernel = _make_sc_kernel()


@jax.jit
def kernel(pos_h, pos_t, pos_r, pos_type_r, neg_h, neg_t, neg_r,
           ent_embeddings, rel_embeddings, type_transfer_matrix):
    parts = _sc_kernel(pos_h, pos_t, pos_r, pos_type_r, neg_h, neg_t, neg_r,
                       ent_embeddings, rel_embeddings, type_transfer_matrix)
    return jnp.sum(parts[:, 0]) / jnp.float32(B)


# speedup vs baseline: 1.8719x; 1.8719x over previous
import functools
import jax
import jax.numpy as jnp
from jax import lax
from jax.experimental import pallas as pl
from jax.experimental.pallas import tpu as pltpu
from jax.experimental.pallas import tpu_sc as plsc

B = 16384
D = 64
NC = 2
NS = 16
NW = NC * NS
EPW = B // NW
SUB = 8
NSUB = EPW // SUB
LANES = 16
NJC = D // LANES


def _rsqrt_newton(x):
    xi = lax.bitcast_convert_type(x, jnp.int32)
    yi = jnp.full((LANES,), 0x5F3759DF, jnp.int32) - lax.shift_right_arithmetic(
        xi, jnp.full((LANES,), 1, jnp.int32))
    y = lax.bitcast_convert_type(yi, jnp.float32)
    xh = x * jnp.float32(0.5)
    for _ in range(3):
        y = y * (jnp.float32(1.5) - xh * y * y)
    return y


def _make_sc_kernel():
    mesh = plsc.VectorSubcoreMesh(core_axis_name="c", subcore_axis_name="s")

    @functools.partial(
        pl.kernel,
        out_type=jax.ShapeDtypeStruct((NW, LANES), jnp.float32),
        mesh=mesh,
        compiler_params=pltpu.CompilerParams(use_tc_tiling_on_sc=False),
        scratch_types=[
            pltpu.VMEM((7 * EPW,), jnp.int32),
            pltpu.VMEM((6 * SUB, D), jnp.float32),
            pltpu.VMEM((6 * SUB, D), jnp.float32),
            pltpu.VMEM((SUB, D * D), jnp.float32),
            pltpu.VMEM((SUB, D * D), jnp.float32),
            pltpu.VMEM((LANES,), jnp.float32),
            pltpu.SemaphoreType.DMA,
            pltpu.SemaphoreType.DMA,
        ],
    )
    def k(pos_h, pos_t, pos_r, pos_type_r, neg_h, neg_t, neg_r,
          ent, rel, mat, out_hbm,
          idx, rowsA, rowsB, mbufA, mbufB, outv, semA, semB):
        wid = lax.axis_index("s") * NC + lax.axis_index("c")
        base = wid * EPW

        for a, src in enumerate((pos_h, pos_t, neg_h, neg_t,
                                 pos_r, neg_r, pos_type_r)):
            pltpu.sync_copy(src.at[pl.ds(base, EPW)],
                            idx.at[pl.ds(a * EPW, EPW)])

        def transfers(rows, mbuf, sem, off):
            out = []
            for a, table in enumerate((ent, ent, ent, ent, rel, rel)):
                out.append(pltpu.make_async_copy(
                    table.at[idx.at[pl.ds(a * EPW + off, SUB)]],
                    rows.at[pl.ds(a * SUB, SUB)], sem))
            out.append(pltpu.make_async_copy(
                mat.at[idx.at[pl.ds(6 * EPW + off, SUB)]], mbuf, sem))
            return out

        def fire(rows, mbuf, sem, off):
            for t in transfers(rows, mbuf, sem, off):
                t.start()

        def drain(rows, mbuf, sem, off):
            for t in transfers(rows, mbuf, sem, off):
                t.wait()

        lane = lax.iota(jnp.int32, LANES)
        perms = [lane ^ k for k in (8, 4, 2, 1)]

        def allsum(v):
            for p in perms:
                v = v + v.at[p].get(mode="promise_in_bounds",
                                    unique_indices=True)
            return v

        def norm_scale(chunks):
            ss = chunks[0] * chunks[0]
            for c in chunks[1:]:
                ss = ss + c * c
            tot = allsum(ss)
            return _rsqrt_newton(jnp.maximum(tot, jnp.float32(1e-12)))

        def element(rows, mbuf, e):
            zero = jnp.zeros((LANES,), jnp.float32)
            a0 = [zero] * NJC
            a1 = [zero] * NJC
            a2 = [zero] * NJC
            a3 = [zero] * NJC
            for ci in range(D // LANES):
                hch = rows[e, pl.ds(ci * LANES, LANES)]
                tch = rows[SUB + e, pl.ds(ci * LANES, LANES)]
                nhch = rows[2 * SUB + e, pl.ds(ci * LANES, LANES)]
                ntch = rows[3 * SUB + e, pl.ds(ci * LANES, LANES)]
                for li in range(LANES):
                    sh = hch[li]
                    st = tch[li]
                    snh = nhch[li]
                    snt = ntch[li]
                    moff = ci * (LANES * D) + li * D
                    for jc in range(NJC):
                        m = mbuf[e, pl.ds(moff + jc * LANES, LANES)]
                        a0[jc] = a0[jc] + sh * m
                        a1[jc] = a1[jc] + st * m
                        a2[jc] = a2[jc] + snh * m
                        a3[jc] = a3[jc] + snt * m
            h_, t_, nh_, nt_ = tuple(a0), tuple(a1), tuple(a2), tuple(a3)

            pr = tuple(rows[4 * SUB + e, pl.ds(jc * LANES, LANES)]
                       for jc in range(NJC))
            nr = tuple(rows[5 * SUB + e, pl.ds(jc * LANES, LANES)]
                       for jc in range(NJC))

            a_h = norm_scale(h_)
            a_t = norm_scale(t_)
            a_nh = norm_scale(nh_)
            a_nt = norm_scale(nt_)
            a_pr = norm_scale(pr)
            a_nr = norm_scale(nr)

            ps = jnp.zeros((LANES,), jnp.float32)
            ns = jnp.zeros((LANES,), jnp.float32)
            for jc in range(NJC):
                ps = ps + jnp.abs(h_[jc] * a_h + pr[jc] * a_pr - t_[jc] * a_t)
                ns = ns + jnp.abs(nh_[jc] * a_nh + nr[jc] * a_nr - nt_[jc] * a_nt)
            diff = allsum(ps - ns)
            return jnp.maximum(diff + jnp.float32(1.0), jnp.float32(0.0))

        def compute(rows, mbuf, loss_acc):
            return lax.fori_loop(
                0, SUB, lambda e, acc: acc + element(rows, mbuf, e), loss_acc)

        last_off = (NSUB - 1) * SUB
        fire(rowsA, mbufA, semA, 0)

        def pair_body(c, loss_acc):
            offA = pl.multiple_of(2 * c * SUB, SUB)
            offB = pl.multiple_of((2 * c + 1) * SUB, SUB)
            offA2 = pl.multiple_of(
                jnp.minimum((2 * c + 2) * SUB, last_off), SUB)
            fire(rowsB, mbufB, semB, offB)
            drain(rowsA, mbufA, semA, offA)
            loss_acc = compute(rowsA, mbufA, loss_acc)
            fire(rowsA, mbufA, semA, offA2)
            drain(rowsB, mbufB, semB, offB)
            return compute(rowsB, mbufB, loss_acc)

        loss = lax.fori_loop(0, NSUB // 2, pair_body,
                             jnp.zeros((LANES,), jnp.float32))
        drain(rowsA, mbufA, semA, last_off)

        outv[...] = loss
        pltpu.sync_copy(outv, out_hbm.at[wid])

    return k


_sc_kernel = _make_sc_kernel()


@jax.jit
def kernel(pos_h, pos_t, pos_r, pos_type_r, neg_h, neg_t, neg_r,
           ent_embeddings, rel_embeddings, type_transfer_matrix):
    parts = _sc_kernel(pos_h, pos_t, pos_r, pos_type_r, neg_h, neg_t, neg_r,
                       ent_embeddings, rel_embeddings, type_transfer_matrix)
    return jnp.sum(parts[:, 0]) / jnp.float32(B)


# speedup vs baseline: 1.8934x; 1.0115x over previous
import functools
import jax
import jax.numpy as jnp
from jax import lax
from jax.experimental import pallas as pl
from jax.experimental.pallas import tpu as pltpu
from jax.experimental.pallas import tpu_sc as plsc

B = 16384
D = 64
NC = 2
NS = 16
NW = NC * NS
EPW = B // NW
SUB = 8
NSUB = EPW // SUB
LANES = 16
NJC = D // LANES


def _rsqrt_newton(x):
    xi = lax.bitcast_convert_type(x, jnp.int32)
    yi = jnp.full((LANES,), 0x5F3759DF, jnp.int32) - lax.shift_right_arithmetic(
        xi, jnp.full((LANES,), 1, jnp.int32))
    y = lax.bitcast_convert_type(yi, jnp.float32)
    xh = x * jnp.float32(0.5)
    for _ in range(2):
        y = y * (jnp.float32(1.5) - xh * y * y)
    return y


def _make_sc_kernel():
    mesh = plsc.VectorSubcoreMesh(core_axis_name="c", subcore_axis_name="s")

    @functools.partial(
        pl.kernel,
        out_type=jax.ShapeDtypeStruct((NW, LANES), jnp.float32),
        mesh=mesh,
        compiler_params=pltpu.CompilerParams(use_tc_tiling_on_sc=False),
        scratch_types=[
            pltpu.VMEM((7 * EPW,), jnp.int32),
            pltpu.VMEM((6 * SUB, D), jnp.float32),
            pltpu.VMEM((6 * SUB, D), jnp.float32),
            pltpu.VMEM((SUB, D * D), jnp.float32),
            pltpu.VMEM((SUB, D * D), jnp.float32),
            pltpu.VMEM((LANES,), jnp.float32),
            pltpu.SemaphoreType.DMA,
            pltpu.SemaphoreType.DMA,
        ],
    )
    def k(pos_h, pos_t, pos_r, pos_type_r, neg_h, neg_t, neg_r,
          ent, rel, mat, out_hbm,
          idx, rowsA, rowsB, mbufA, mbufB, outv, semA, semB):
        wid = lax.axis_index("s") * NC + lax.axis_index("c")
        base = wid * EPW

        for a, src in enumerate((pos_h, pos_t, neg_h, neg_t,
                                 pos_r, neg_r, pos_type_r)):
            pltpu.sync_copy(src.at[pl.ds(base, EPW)],
                            idx.at[pl.ds(a * EPW, EPW)])

        def transfers(rows, mbuf, sem, off):
            out = []
            for a, table in enumerate((ent, ent, ent, ent, rel, rel)):
                out.append(pltpu.make_async_copy(
                    table.at[idx.at[pl.ds(a * EPW + off, SUB)]],
                    rows.at[pl.ds(a * SUB, SUB)], sem))
            out.append(pltpu.make_async_copy(
                mat.at[idx.at[pl.ds(6 * EPW + off, SUB)]], mbuf, sem))
            return out

        def fire(rows, mbuf, sem, off):
            for t in transfers(rows, mbuf, sem, off):
                t.start()

        def drain(rows, mbuf, sem, off):
            for t in transfers(rows, mbuf, sem, off):
                t.wait()

        lane = lax.iota(jnp.int32, LANES)
        perms = [lane ^ k for k in (8, 4, 2, 1)]

        def allsum(v):
            for p in perms:
                v = v + v.at[p].get(mode="promise_in_bounds",
                                    unique_indices=True)
            return v

        def norm_scale(chunks):
            ss = chunks[0] * chunks[0]
            for c in chunks[1:]:
                ss = ss + c * c
            tot = allsum(ss)
            return _rsqrt_newton(jnp.maximum(tot, jnp.float32(1e-12)))

        def element(rows, mbuf, e):
            zero = jnp.zeros((LANES,), jnp.float32)
            a0 = [zero] * NJC
            a1 = [zero] * NJC
            a2 = [zero] * NJC
            a3 = [zero] * NJC
            for ci in range(D // LANES):
                hch = rows[e, pl.ds(ci * LANES, LANES)]
                tch = rows[SUB + e, pl.ds(ci * LANES, LANES)]
                nhch = rows[2 * SUB + e, pl.ds(ci * LANES, LANES)]
                ntch = rows[3 * SUB + e, pl.ds(ci * LANES, LANES)]
                for li in range(LANES):
                    sh = hch[li]
                    st = tch[li]
                    snh = nhch[li]
                    snt = ntch[li]
                    moff = ci * (LANES * D) + li * D
                    for jc in range(NJC):
                        m = mbuf[e, pl.ds(moff + jc * LANES, LANES)]
                        a0[jc] = a0[jc] + sh * m
                        a1[jc] = a1[jc] + st * m
                        a2[jc] = a2[jc] + snh * m
                        a3[jc] = a3[jc] + snt * m
            h_, t_, nh_, nt_ = tuple(a0), tuple(a1), tuple(a2), tuple(a3)

            pr = tuple(rows[4 * SUB + e, pl.ds(jc * LANES, LANES)]
                       for jc in range(NJC))
            nr = tuple(rows[5 * SUB + e, pl.ds(jc * LANES, LANES)]
                       for jc in range(NJC))

            a_h = norm_scale(h_)
            a_t = norm_scale(t_)
            a_nh = norm_scale(nh_)
            a_nt = norm_scale(nt_)

            ps = jnp.zeros((LANES,), jnp.float32)
            ns = jnp.zeros((LANES,), jnp.float32)
            for jc in range(NJC):
                ps = ps + jnp.abs(h_[jc] * a_h + pr[jc] - t_[jc] * a_t)
                ns = ns + jnp.abs(nh_[jc] * a_nh + nr[jc] - nt_[jc] * a_nt)
            diff = allsum(ps - ns)
            return jnp.maximum(diff + jnp.float32(1.0), jnp.float32(0.0))

        def compute(rows, mbuf, loss_acc):
            return lax.fori_loop(
                0, SUB, lambda e, acc: acc + element(rows, mbuf, e), loss_acc)

        last_off = (NSUB - 1) * SUB
        fire(rowsA, mbufA, semA, 0)

        def pair_body(c, loss_acc):
            plsc.subcore_barrier()
            offA = pl.multiple_of(2 * c * SUB, SUB)
            offB = pl.multiple_of((2 * c + 1) * SUB, SUB)
            offA2 = pl.multiple_of(
                jnp.minimum((2 * c + 2) * SUB, last_off), SUB)
            fire(rowsB, mbufB, semB, offB)
            drain(rowsA, mbufA, semA, offA)
            loss_acc = compute(rowsA, mbufA, loss_acc)
            fire(rowsA, mbufA, semA, offA2)
            drain(rowsB, mbufB, semB, offB)
            return compute(rowsB, mbufB, loss_acc)

        loss = lax.fori_loop(0, NSUB // 2, pair_body,
                             jnp.zeros((LANES,), jnp.float32))
        drain(rowsA, mbufA, semA, last_off)

        outv[...] = loss
        pltpu.sync_copy(outv, out_hbm.at[wid])

    return k


_sc_kernel = _make_sc_kernel()


@jax.jit
def kernel(pos_h, pos_t, pos_r, pos_type_r, neg_h, neg_t, neg_r,
           ent_embeddings, rel_embeddings, type_transfer_matrix):
    sq = jnp.sum(jnp.square(rel_embeddings), axis=-1, keepdims=True)
    rel_n = rel_embeddings * jax.lax.rsqrt(jnp.maximum(sq, 1e-12))
    parts = _sc_kernel(pos_h, pos_t, pos_r, pos_type_r, neg_h, neg_t, neg_r,
                       ent_embeddings, rel_n, type_transfer_matrix)
    return jnp.sum(parts[:, 0]) / jnp.float32(B)
